# baseline (speedup 1.0000x reference)
import functools

import jax
import jax.numpy as jnp
from jax import lax
from jax.experimental import pallas as pl
from jax.experimental.pallas import tpu as pltpu
from jax.experimental.pallas import tpu_sc as plsc

_BATCH = 4
_SEQ = 8192
_D = 1024
_ROWS = _BATCH * _SEQ
_C = 32
_TOTAL_CHUNKS = _ROWS // _C
_NBUF = 2


@functools.partial(jax.jit, static_argnums=(2, 3))
def _sc_gather(ids2d, table, nc, ns):
    nw = nc * ns
    ch_w = _TOTAL_CHUNKS // nw
    assert ch_w % 4 == 0 and ch_w >= 8

    mesh = plsc.VectorSubcoreMesh(core_axis_name="c", subcore_axis_name="s")

    @functools.partial(
        pl.kernel,
        mesh=mesh,
        out_type=jax.ShapeDtypeStruct((_ROWS, _D), jnp.float32),
        scratch_types=[
            pltpu.VMEM((ch_w, _C), jnp.int32),
            pltpu.VMEM((_NBUF, _C, _D), jnp.float32),
            pltpu.VMEM_SHARED((ns, _C, _D), jnp.float32),
            pltpu.SemaphoreType.DMA,
            pltpu.SemaphoreType.DMA,
            pltpu.SemaphoreType.DMA,
            pltpu.SemaphoreType.DMA,
            pltpu.SemaphoreType.DMA,
            pltpu.SemaphoreType.DMA,
        ],
    )
    def k(ids_hbm, table_hbm, out_hbm, idx_v, bufs, sh, g0, g1, a0, a1, b0, b1):
        gsem = (g0, g1)
        asem = (a0, a1)
        bsem = (b0, b1)
        sid = lax.axis_index("s")
        wid = sid * nc + lax.axis_index("c")
        base_chunk = wid * ch_w
        pltpu.sync_copy(ids_hbm.at[pl.ds(base_chunk, ch_w)], idx_v)

        def out_slice(c):
            return out_hbm.at[pl.ds((base_chunk + c) * _C, _C)]

        def gather(c, b):
            return pltpu.make_async_copy(
                table_hbm.at[idx_v.at[c]], bufs.at[b], gsem[b])

        def scat_a(c, b):
            return pltpu.make_async_copy(bufs.at[b], out_slice(c), asem[b])

        def scat_b(c):
            return pltpu.make_async_copy(sh.at[sid], out_slice(c), bsem[0])

        def slot(c, b, rb=None, first=False, last=False):
            gather(c, b).wait()
            if rb is None:
                scat_a(c, b).start()
                scat_a(c, b).wait()
            else:
                if not first:
                    scat_b(c - 2).wait()
                pltpu.sync_copy(bufs.at[b], sh.at[sid])
                scat_b(c).start()
            if not last:
                gather(c + 2, b).start()

        gather(0, 0).start()
        gather(1, 1).start()
        slot(0, 0)
        slot(1, 1, rb=0, first=True)
        slot(2, 0)
        slot(3, 1, rb=0)

        def body(g, carry):
            for bb in range(4):
                slot(4 + g * 4 + bb, bb % 2, rb=None if bb % 2 == 0 else (bb // 2))
            return carry

        lax.fori_loop(0, (ch_w - 8) // 4, body, 0)

        slot(ch_w - 4, 0)
        slot(ch_w - 3, 1, rb=0)
        slot(ch_w - 2, 0, last=True)
        slot(ch_w - 1, 1, rb=0, last=True)
        scat_b(ch_w - 1).wait()

    return k(ids2d, table)


def kernel(position_ids, table):
    info = plsc.get_sparse_core_info()
    ids2d = position_ids.reshape(_TOTAL_CHUNKS, _C)
    out = _sc_gather(ids2d, table, int(info.num_cores), int(info.num_subcores))
    return out.reshape(_BATCH, _SEQ, _D)


# speedup vs baseline: 1.0069x; 1.0069x over previous
import functools

import jax
import jax.numpy as jnp
from jax import lax
from jax.experimental import pallas as pl
from jax.experimental.pallas import tpu as pltpu
from jax.experimental.pallas import tpu_sc as plsc

_BATCH = 4
_SEQ = 8192
_D = 1024
_ROWS = _BATCH * _SEQ
_C = 32
_TOTAL_CHUNKS = _ROWS // _C
_NBUF = 2


@functools.partial(jax.jit, static_argnums=(2, 3))
def _sc_gather(ids2d, table, nc, ns):
    nw = nc * ns
    ch_w = _TOTAL_CHUNKS // nw
    assert ch_w % _NBUF == 0 and ch_w >= 2 * _NBUF

    mesh = plsc.VectorSubcoreMesh(core_axis_name="c", subcore_axis_name="s")

    @functools.partial(
        pl.kernel,
        mesh=mesh,
        out_type=jax.ShapeDtypeStruct((_ROWS, _D), jnp.float32),
        scratch_types=[
            pltpu.VMEM((ch_w, _C), jnp.int32),
            pltpu.VMEM((_NBUF, _C, _D), jnp.float32),
            pltpu.SemaphoreType.DMA,
            pltpu.SemaphoreType.DMA,
            pltpu.SemaphoreType.DMA,
            pltpu.SemaphoreType.DMA,
        ],
    )
    def k(ids_hbm, table_hbm, out_hbm, idx_v, bufs, g0, g1, s0, s1):
        gsem = (g0, g1)
        ssem = (s0, s1)
        wid = lax.axis_index("s") * nc + lax.axis_index("c")
        base_chunk = wid * ch_w
        pltpu.sync_copy(ids_hbm.at[pl.ds(base_chunk, ch_w)], idx_v)

        def gather(c, b):
            return pltpu.make_async_copy(
                table_hbm.at[idx_v.at[c]], bufs.at[b], gsem[b])

        def scatter(c, b):
            return pltpu.make_async_copy(
                bufs.at[b], out_hbm.at[pl.ds((base_chunk + c) * _C, _C)],
                ssem[b])

        for b in range(_NBUF):
            gather(b, b).start()

        def pair_body(g, carry):
            for b in range(_NBUF):
                c = g * _NBUF + b
                gather(c, b).wait()
                scatter(c, b).start()
                scatter(c, b).wait()
                gather(c + _NBUF, b).start()
            return carry

        lax.fori_loop(0, ch_w // _NBUF - 1, pair_body, 0)

        for b in range(_NBUF):
            c = ch_w - _NBUF + b
            gather(c, b).wait()
            scatter(c, b).start()
            scatter(c, b).wait()

    return k(ids2d, table)


def kernel(position_ids, table):
    info = plsc.get_sparse_core_info()
    ids2d = position_ids.reshape(_TOTAL_CHUNKS, _C)
    out = _sc_gather(ids2d, table, int(info.num_cores), int(info.num_subcores))
    return out.reshape(_BATCH, _SEQ, _D)
